# baseline (speedup 1.0000x reference)
import jax
import jax.numpy as jnp
from jax.experimental import pallas as pl
from jax.experimental.pallas import tpu as pltpu

_B, _S, _D, _F, _E, _K = 1, 2048, 768, 1536, 8, 2


def _moe_kernel(res_ref, rw_ref, win_ref, bin_ref, wout_ref, bout_ref,
                out_ref, resbf_ref, i1_ref, i2_ref, p1_ref, p2_ref):
    e = pl.program_id(0)

    @pl.when(e == 0)
    def _routing():
        x = res_ref[0]
        resbf_ref[...] = x.astype(jnp.bfloat16)
        logits = jnp.dot(x.astype(jnp.bfloat16), rw_ref[...].T.astype(jnp.bfloat16),
                         preferred_element_type=jnp.float32)
        m = jnp.max(logits, axis=-1, keepdims=True)
        ex = jnp.exp(logits - m)
        probs = ex / jnp.sum(ex, axis=-1, keepdims=True)
        idx = jax.lax.broadcasted_iota(jnp.int32, probs.shape, 1)
        p1 = jnp.max(probs, axis=-1, keepdims=True)
        i1 = jnp.min(jnp.where(probs >= p1, idx, _E), axis=-1, keepdims=True)
        probs2 = jnp.where(idx == i1, -1.0, probs)
        p2 = jnp.max(probs2, axis=-1, keepdims=True)
        i2 = jnp.min(jnp.where(probs2 >= p2, idx, _E), axis=-1, keepdims=True)
        denom = p1 + p2 + 1e-8
        p1_ref[...] = p1 / denom
        p2_ref[...] = p2 / denom
        i1_ref[...] = i1
        i2_ref[...] = i2
        out_ref[...] = jnp.zeros_like(out_ref)

    win_bf = win_ref[0].astype(jnp.bfloat16)
    wout_bf = wout_ref[0].astype(jnp.bfloat16)
    nchunk = 2
    cs = _S // nchunk
    for c in range(nchunk):
        sl = pl.ds(c * cs, cs)
        wcol = (jnp.where(i1_ref[sl, :] == e, p1_ref[sl, :], 0.0)
                + jnp.where(i2_ref[sl, :] == e, p2_ref[sl, :], 0.0))
        xbf = resbf_ref[sl, :]
        h = jnp.dot(xbf, win_bf,
                    preferred_element_type=jnp.float32).astype(jnp.bfloat16)
        h = h + bin_ref[0].astype(jnp.bfloat16)
        g2 = h * (jnp.bfloat16(1.0)
                  + jax.lax.erf(h * jnp.bfloat16(0.70710678)))
        o2 = jnp.dot(g2, wout_bf, preferred_element_type=jnp.float32)
        out_ref[0, sl, :] += o2 * (0.5 * wcol) + bout_ref[0] * wcol


def kernel(residual, router_w, W_in, b_in, W_out, b_out):
    S, D, F, E = _S, _D, _F, _E
    out = pl.pallas_call(
        _moe_kernel,
        grid=(E,),
        in_specs=[
            pl.BlockSpec((1, S, D), lambda e: (0, 0, 0)),
            pl.BlockSpec((E, D), lambda e: (0, 0)),
            pl.BlockSpec((1, D, F), lambda e: (e, 0, 0)),
            pl.BlockSpec((1, 1, F), lambda e: (e, 0, 0)),
            pl.BlockSpec((1, F, D), lambda e: (e, 0, 0)),
            pl.BlockSpec((1, 1, D), lambda e: (e, 0, 0)),
        ],
        out_specs=pl.BlockSpec((1, S, D), lambda e: (0, 0, 0)),
        out_shape=jax.ShapeDtypeStruct((_B, S, D), jnp.float32),
        scratch_shapes=[
            pltpu.VMEM((S, D), jnp.bfloat16),
            pltpu.VMEM((S, 1), jnp.int32),
            pltpu.VMEM((S, 1), jnp.int32),
            pltpu.VMEM((S, 1), jnp.float32),
            pltpu.VMEM((S, 1), jnp.float32),
        ],
    )(residual, router_w, W_in, b_in.reshape(E, 1, F), W_out,
      b_out.reshape(E, 1, D))
    return out


# speedup vs baseline: 1.0026x; 1.0026x over previous
import jax
import jax.numpy as jnp
from jax.experimental import pallas as pl
from jax.experimental.pallas import tpu as pltpu

_B, _S, _D, _F, _E, _K = 1, 2048, 768, 1536, 8, 2


def _moe_kernel(res_ref, rw_ref, win_ref, bin_ref, wout_ref, bout_ref,
                out_ref, resbf_ref, i1_ref, i2_ref, p1_ref, p2_ref):
    e = pl.program_id(0)

    @pl.when(e == 0)
    def _routing():
        x = res_ref[0]
        resbf_ref[...] = x.astype(jnp.bfloat16)
        logits = jnp.dot(x.astype(jnp.bfloat16), rw_ref[...].T.astype(jnp.bfloat16),
                         preferred_element_type=jnp.float32)
        m = jnp.max(logits, axis=-1, keepdims=True)
        ex = jnp.exp(logits - m)
        probs = ex / jnp.sum(ex, axis=-1, keepdims=True)
        idx = jax.lax.broadcasted_iota(jnp.int32, probs.shape, 1)
        p1 = jnp.max(probs, axis=-1, keepdims=True)
        i1 = jnp.min(jnp.where(probs >= p1, idx, _E), axis=-1, keepdims=True)
        probs2 = jnp.where(idx == i1, -1.0, probs)
        p2 = jnp.max(probs2, axis=-1, keepdims=True)
        i2 = jnp.min(jnp.where(probs2 >= p2, idx, _E), axis=-1, keepdims=True)
        denom = p1 + p2 + 1e-8
        p1_ref[...] = p1 / denom
        p2_ref[...] = p2 / denom
        i1_ref[...] = i1
        i2_ref[...] = i2
        out_ref[...] = jnp.zeros_like(out_ref)

    win_bf = win_ref[0].astype(jnp.bfloat16)
    wout_bf = wout_ref[0].astype(jnp.bfloat16)
    nchunk = 4
    cs = _S // nchunk
    for c in range(nchunk):
        sl = pl.ds(c * cs, cs)
        wcol = (jnp.where(i1_ref[sl, :] == e, p1_ref[sl, :], 0.0)
                + jnp.where(i2_ref[sl, :] == e, p2_ref[sl, :], 0.0))
        xbf = resbf_ref[sl, :]
        h = jnp.dot(xbf, win_bf, preferred_element_type=jnp.float32)
        h = h + bin_ref[0]
        g2 = h * (1.0 + jax.lax.erf(h * 0.7071067811865476))
        o2 = jnp.dot(g2.astype(jnp.bfloat16), wout_bf,
                     preferred_element_type=jnp.float32)
        out_ref[0, sl, :] += o2 * (0.5 * wcol) + bout_ref[0] * wcol


def kernel(residual, router_w, W_in, b_in, W_out, b_out):
    S, D, F, E = _S, _D, _F, _E
    out = pl.pallas_call(
        _moe_kernel,
        grid=(E,),
        in_specs=[
            pl.BlockSpec((1, S, D), lambda e: (0, 0, 0)),
            pl.BlockSpec((E, D), lambda e: (0, 0)),
            pl.BlockSpec((1, D, F), lambda e: (e, 0, 0)),
            pl.BlockSpec((1, 1, F), lambda e: (e, 0, 0)),
            pl.BlockSpec((1, F, D), lambda e: (e, 0, 0)),
            pl.BlockSpec((1, 1, D), lambda e: (e, 0, 0)),
        ],
        out_specs=pl.BlockSpec((1, S, D), lambda e: (0, 0, 0)),
        out_shape=jax.ShapeDtypeStruct((_B, S, D), jnp.float32),
        scratch_shapes=[
            pltpu.VMEM((S, D), jnp.bfloat16),
            pltpu.VMEM((S, 1), jnp.int32),
            pltpu.VMEM((S, 1), jnp.int32),
            pltpu.VMEM((S, 1), jnp.float32),
            pltpu.VMEM((S, 1), jnp.float32),
        ],
    )(residual, router_w, W_in, b_in.reshape(E, 1, F), W_out,
      b_out.reshape(E, 1, D))
    return out
